# baseline (speedup 1.0000x reference)
import functools

import jax
import jax.numpy as jnp
from jax import lax
from jax.experimental import pallas as pl
from jax.experimental.pallas import tpu as pltpu
from jax.experimental.pallas import tpu_sc as plsc

DIM = 64
EPS = 1e-5

NC = 2
NS = 16
L = 16
NW = NC * NS

XROWS = 4096
XCOLS = 200
B_TOTAL = XROWS * XCOLS
PER_W = B_TOTAL // NW
XR_W = XROWS // NW
CHUNK = 128
N_CHUNKS = PER_W // CHUNK
GPC = CHUNK // L


_GDN = lax.GatherDimensionNumbers(
    offset_dims=(), collapsed_slice_dims=(0,), start_index_map=(0,))


def _lane_rotate(v, idx):
    return lax.gather(
        v, idx, _GDN, (1,), mode=lax.GatherScatterMode.PROMISE_IN_BOUNDS)


def _rsqrt_nr(x):
    i = plsc.bitcast(x, jnp.int32)
    i = jnp.int32(0x5F3759DF) - (i >> 1)
    y = plsc.bitcast(i, jnp.float32)
    for _ in range(2):
        y = y * (1.5 - 0.5 * x * y * y)
    return y


@functools.cache
def _make_sc_kernel():
    mesh = plsc.VectorSubcoreMesh(
        core_axis_name="c", subcore_axis_name="s",
        num_cores=NC, num_subcores=NS)

    @functools.partial(
        pl.kernel,
        mesh=mesh,
        out_type=jax.ShapeDtypeStruct((B_TOTAL * DIM,), jnp.float32),
        scratch_types=[
            pltpu.VMEM((PER_W,), jnp.int32),
            pltpu.VMEM((CHUNK, DIM), jnp.float32),
            pltpu.VMEM((CHUNK, DIM), jnp.float32),
            pltpu.VMEM((CHUNK * DIM,), jnp.float32),
            pltpu.VMEM((CHUNK * DIM,), jnp.float32),
            pltpu.VMEM((DIM,), jnp.float32),
            pltpu.VMEM((DIM,), jnp.float32),
            pltpu.SemaphoreType.DMA,
            pltpu.SemaphoreType.DMA,
            pltpu.SemaphoreType.DMA,
            pltpu.SemaphoreType.DMA,
        ],
        compiler_params=pltpu.CompilerParams(
            needs_layout_passes=False, use_tc_tiling_on_sc=False),
    )
    def sc_kernel(x_hbm, table_hbm, gamma_hbm, beta_hbm, out_hbm,
                  idxf, rows0, rows1, out0, out1,
                  gam_v, bet_v, gsem0, gsem1, osem0, osem1):
        rows_bufs = (rows0, rows1)
        out_bufs = (out0, out1)
        gsems = (gsem0, gsem1)
        osems = (osem0, osem1)
        wid = lax.axis_index("s") * NC + lax.axis_index("c")
        out_row0 = wid * PER_W
        pltpu.sync_copy(gamma_hbm, gam_v)
        pltpu.sync_copy(beta_hbm, bet_v)
        pltpu.sync_copy(x_hbm.at[pl.ds(wid * PER_W, PER_W)], idxf)

        def fire_gather(ci, buf, sem):
            pltpu.async_copy(
                table_hbm.at[idxf.at[pl.ds(ci * CHUNK, CHUNK)]], buf, sem)

        def wait_gather(ci, buf, sem):
            pltpu.make_async_copy(
                table_hbm.at[idxf.at[pl.ds(ci * CHUNK, CHUNK)]], buf, sem
            ).wait()

        def out_slice(ci):
            return out_hbm.at[
                pl.ds((out_row0 + ci * CHUNK) * DIM, CHUNK * DIM)]

        def compute_chunk(ci, rows_v, out_v):
            lane = lax.iota(jnp.int32, L)
            rots = [((lane + off) & (L - 1)).reshape(L, 1)
                    for off in (8, 4, 2, 1)]
            gb = [(gam_v[pl.ds(k * L, L)], bet_v[pl.ds(k * L, L)])
                  for k in range(DIM // L)]

            @pl.loop(0, GPC)
            def _group(g):
                for r in range(L):
                    row = g * L + r
                    vs = [rows_v[row, pl.ds(k * L, L)]
                          for k in range(DIM // L)]
                    s = (vs[0] + vs[1]) + (vs[2] + vs[3])
                    q = ((vs[0] * vs[0] + vs[1] * vs[1])
                         + (vs[2] * vs[2] + vs[3] * vs[3]))
                    for ri in rots:
                        s = s + _lane_rotate(s, ri)
                        q = q + _lane_rotate(q, ri)
                    mu = s * (1.0 / DIM)
                    var = q * (1.0 / DIM) - mu * mu
                    rs = _rsqrt_nr(var + EPS)
                    for k, (gk, bk) in enumerate(gb):
                        out_v[pl.ds(row * DIM + k * L, L)] = (
                            (vs[k] - mu) * rs * gk + bk)

        fire_gather(0, rows_bufs[0], gsems[0])

        @pl.loop(0, N_CHUNKS, step=2)
        def _chunk(i):
            for b in range(2):
                ci = i + b
                if b == 0:
                    fire_gather(i + 1, rows_bufs[1], gsems[1])
                else:
                    @pl.when(i < N_CHUNKS - 2)
                    def _():
                        fire_gather(i + 2, rows_bufs[0], gsems[0])
                wait_gather(ci, rows_bufs[b], gsems[b])

                @pl.when(ci >= 2)
                def _():
                    pltpu.make_async_copy(
                        out_bufs[b], out_slice(ci - 2), osems[b]).wait()

                compute_chunk(ci, rows_bufs[b], out_bufs[b])
                pltpu.async_copy(out_bufs[b], out_slice(ci), osems[b])

        pltpu.make_async_copy(
            out_bufs[0], out_slice(N_CHUNKS - 2), osems[0]).wait()
        pltpu.make_async_copy(
            out_bufs[1], out_slice(N_CHUNKS - 1), osems[1]).wait()

    return sc_kernel


def kernel(x, table, gamma, beta):
    xf = x.reshape(B_TOTAL).astype(jnp.int32)
    out = _make_sc_kernel()(xf, table, gamma, beta)
    return out.reshape(x.shape[0], x.shape[1], DIM)


# speedup vs baseline: 1.0005x; 1.0005x over previous
import functools

import jax
import jax.numpy as jnp
from jax import lax
from jax.experimental import pallas as pl
from jax.experimental.pallas import tpu as pltpu
from jax.experimental.pallas import tpu_sc as plsc

DIM = 64
EPS = 1e-5

NC = 2
NS = 16
L = 16
NW = NC * NS

XROWS = 4096
XCOLS = 200
B_TOTAL = XROWS * XCOLS
PER_W = B_TOTAL // NW
XR_W = XROWS // NW
CHUNK = 128
N_CHUNKS = PER_W // CHUNK
GPC = CHUNK // L


_GDN = lax.GatherDimensionNumbers(
    offset_dims=(), collapsed_slice_dims=(0,), start_index_map=(0,))


def _lane_rotate(v, idx):
    return lax.gather(
        v, idx, _GDN, (1,), mode=lax.GatherScatterMode.PROMISE_IN_BOUNDS)


def _rsqrt_nr(x):
    i = plsc.bitcast(x, jnp.int32)
    i = jnp.int32(0x5F3759DF) - (i >> 1)
    y = plsc.bitcast(i, jnp.float32)
    for _ in range(2):
        y = y * (1.5 - 0.5 * x * y * y)
    return y


@functools.cache
def _make_sc_kernel():
    mesh = plsc.VectorSubcoreMesh(
        core_axis_name="c", subcore_axis_name="s",
        num_cores=NC, num_subcores=NS)

    @functools.partial(
        pl.kernel,
        mesh=mesh,
        out_type=jax.ShapeDtypeStruct((B_TOTAL * DIM // 128, 128),
                                      jnp.float32),
        scratch_types=[
            pltpu.VMEM((PER_W,), jnp.int32),
            pltpu.VMEM((CHUNK, DIM), jnp.float32),
            pltpu.VMEM((CHUNK, DIM), jnp.float32),
            pltpu.VMEM((CHUNK * DIM // 128, 128), jnp.float32),
            pltpu.VMEM((CHUNK * DIM // 128, 128), jnp.float32),
            pltpu.VMEM((DIM,), jnp.float32),
            pltpu.VMEM((DIM,), jnp.float32),
            pltpu.SemaphoreType.DMA,
            pltpu.SemaphoreType.DMA,
            pltpu.SemaphoreType.DMA,
            pltpu.SemaphoreType.DMA,
        ],
        compiler_params=pltpu.CompilerParams(
            needs_layout_passes=False, use_tc_tiling_on_sc=False),
    )
    def sc_kernel(x_hbm, table_hbm, gamma_hbm, beta_hbm, out_hbm,
                  idxf, rows0, rows1, out0, out1,
                  gam_v, bet_v, gsem0, gsem1, osem0, osem1):
        rows_bufs = (rows0, rows1)
        out_bufs = (out0, out1)
        gsems = (gsem0, gsem1)
        osems = (osem0, osem1)
        wid = lax.axis_index("s") * NC + lax.axis_index("c")
        out_row0 = wid * PER_W
        pltpu.sync_copy(gamma_hbm, gam_v)
        pltpu.sync_copy(beta_hbm, bet_v)
        pltpu.sync_copy(x_hbm.at[pl.ds(wid * PER_W, PER_W)], idxf)

        def fire_gather(ci, buf, sem):
            pltpu.async_copy(
                table_hbm.at[idxf.at[pl.ds(ci * CHUNK, CHUNK)]], buf, sem)

        def wait_gather(ci, buf, sem):
            pltpu.make_async_copy(
                table_hbm.at[idxf.at[pl.ds(ci * CHUNK, CHUNK)]], buf, sem
            ).wait()

        def out_slice(ci):
            return out_hbm.at[pl.ds(
                (out_row0 + ci * CHUNK) * DIM // 128, CHUNK * DIM // 128)]

        def compute_chunk(ci, rows_v, out_v):
            lane = lax.iota(jnp.int32, L)
            rots = [((lane + off) & (L - 1)).reshape(L, 1)
                    for off in (8, 4, 2, 1)]
            gb = [(gam_v[pl.ds(k * L, L)], bet_v[pl.ds(k * L, L)])
                  for k in range(DIM // L)]

            @pl.loop(0, GPC)
            def _group(g):
                for r in range(L):
                    row = g * L + r
                    vs = [rows_v[row, pl.ds(k * L, L)]
                          for k in range(DIM // L)]
                    s = (vs[0] + vs[1]) + (vs[2] + vs[3])
                    q = ((vs[0] * vs[0] + vs[1] * vs[1])
                         + (vs[2] * vs[2] + vs[3] * vs[3]))
                    for ri in rots:
                        s = s + _lane_rotate(s, ri)
                        q = q + _lane_rotate(q, ri)
                    mu = s * (1.0 / DIM)
                    var = q * (1.0 / DIM) - mu * mu
                    rs = _rsqrt_nr(var + EPS)
                    orow = g * (L // 2) + (r >> 1)
                    for k, (gk, bk) in enumerate(gb):
                        oc = (r & 1) * DIM + k * L
                        out_v[orow, pl.ds(oc, L)] = (
                            (vs[k] - mu) * rs * gk + bk)

        fire_gather(0, rows_bufs[0], gsems[0])

        @pl.loop(0, N_CHUNKS, step=2)
        def _chunk(i):
            for b in range(2):
                ci = i + b
                if b == 0:
                    fire_gather(i + 1, rows_bufs[1], gsems[1])
                else:
                    @pl.when(i < N_CHUNKS - 2)
                    def _():
                        fire_gather(i + 2, rows_bufs[0], gsems[0])
                wait_gather(ci, rows_bufs[b], gsems[b])

                @pl.when(ci >= 2)
                def _():
                    pltpu.make_async_copy(
                        out_bufs[b], out_slice(ci - 2), osems[b]).wait()

                compute_chunk(ci, rows_bufs[b], out_bufs[b])
                pltpu.async_copy(out_bufs[b], out_slice(ci), osems[b])

        pltpu.make_async_copy(
            out_bufs[0], out_slice(N_CHUNKS - 2), osems[0]).wait()
        pltpu.make_async_copy(
            out_bufs[1], out_slice(N_CHUNKS - 1), osems[1]).wait()

    return sc_kernel


def kernel(x, table, gamma, beta):
    xf = x.reshape(B_TOTAL).astype(jnp.int32)
    out = _make_sc_kernel()(xf, table, gamma, beta)
    return out.reshape(x.shape[0], x.shape[1], DIM)
